# baseline (speedup 1.0000x reference)
import functools

import jax
import jax.numpy as jnp
from jax import lax
from jax.experimental import pallas as pl
from jax.experimental.pallas import tpu as pltpu
from jax.experimental.pallas import tpu_sc as plsc

N_NODES = 100000
N_PAD = 102400
DUMMY = N_NODES
LANES = 16
GR = 512
E_ROWS = 3200
ZROWS = 1600
R_TC = 2048
NEG = -1e30


def _make_scatter_kernel(mode):
    col_split = mode == "agg32"
    n_workers = 16 if col_split else 32
    rows_per_tile = E_ROWS // n_workers
    groups = rows_per_tile
    mesh = plsc.VectorSubcoreMesh(core_axis_name="c", subcore_axis_name="s")

    def body(table, srcr, dstr, e0c, out, src_v, dst_v, dbuf, accum,
             sem_g, sem_s):
        cid = lax.axis_index("c")
        sid = lax.axis_index("s")
        tix = cid if col_split else 0
        for t in range(4):
            pltpu.sync_copy(table.at[tix, pl.ds(N_NODES, ZROWS)],
                            accum.at[pl.ds(sid * 6400 + t * ZROWS, ZROWS)])
        if mode == "deg":
            for j in range(GR // 128):
                pltpu.sync_copy(e0c, dbuf.at[0, pl.ds(j * 128, 128)])
        plsc.subcore_barrier()

        if col_split:
            row0 = sid * rows_per_tile
        else:
            row0 = (cid * 16 + sid) * rows_per_tile

        def load_idx(b, g):
            base = (row0 + g) * GR
            if mode != "deg":
                pltpu.sync_copy(srcr.at[pl.ds(base, GR)], src_v.at[b])
            pltpu.sync_copy(dstr.at[pl.ds(base, GR)], dst_v.at[b])

        def fire_gathers(b):
            pltpu.async_copy(table.at[tix].at[src_v.at[b]], dbuf.at[b], sem_g)

        def scatter_src(b):
            return dbuf.at[0] if mode == "deg" else dbuf.at[b]

        load_idx(0, 0)
        if mode != "deg":
            fire_gathers(0)

        def grp(g, carry):
            b = lax.rem(g, 2)
            nb = 1 - b
            if mode != "deg":
                pltpu.make_async_copy(table.at[tix].at[src_v.at[b]],
                                      dbuf.at[b], sem_g).wait()
            pltpu.async_copy(scatter_src(b), accum.at[dst_v.at[b]],
                             sem_s, add=True)

            @pl.when(g >= 1)
            def _drain_prev():
                pltpu.make_async_copy(scatter_src(nb),
                                      accum.at[dst_v.at[nb]], sem_s).wait()

            @pl.when(g + 1 < groups)
            def _prefetch():
                load_idx(nb, g + 1)
                if mode != "deg":
                    fire_gathers(nb)

            return carry

        lax.fori_loop(0, groups, grp, 0)
        last = (groups - 1) % 2
        pltpu.make_async_copy(scatter_src(last),
                              accum.at[dst_v.at[last]], sem_s).wait()
        plsc.subcore_barrier()
        for t in range(4):
            sl = pl.ds(sid * 6400 + t * ZROWS, ZROWS)
            pltpu.sync_copy(accum.at[sl], out.at[cid, sl])

    return pl.kernel(
        body,
        out_type=jax.ShapeDtypeStruct((2, N_PAD, LANES), jnp.float32),
        mesh=mesh,
        scratch_types=[
            pltpu.VMEM((2, GR), jnp.int32),
            pltpu.VMEM((2, GR), jnp.int32),
            pltpu.VMEM((2, GR, LANES), jnp.float32),
            pltpu.VMEM_SHARED((N_PAD, LANES), jnp.float32),
            pltpu.SemaphoreType.DMA,
            pltpu.SemaphoreType.DMA,
        ],
        compiler_params=pltpu.CompilerParams(use_tc_tiling_on_sc=False),
    )


def _row_mask(i):
    rid = lax.broadcasted_iota(jnp.int32, (R_TC, 1), 0) + i * R_TC
    return rid < N_NODES


def _dot(a, b):
    return lax.dot_general(a, b, (((1,), (0,)), ((), ())),
                           preferred_element_type=jnp.float32)


def _k1_body(zd_ref, x_ref, y0_ref, dinv_ref):
    zd = zd_ref[...]
    deg = zd[0, :, 0:1] + zd[1, :, 0:1] + 1.0
    dinv = 1.0 / jnp.sqrt(deg)
    y0_ref[...] = x_ref[...] * dinv
    dinv_ref[...] = jnp.broadcast_to(dinv, dinv_ref.shape)


def _k2_body(z0_ref, y0_ref, dinv_ref, w1_ref, b1_ref, y1_ref):
    i = pl.program_id(0)
    z = z0_ref[...]
    dinv = dinv_ref[...]
    u = (z[0] + z[1] + y0_ref[...]) * dinv
    h = jnp.maximum(_dot(u, w1_ref[...]) + b1_ref[...], 0.0)
    y1 = h * dinv[:, 0:1]
    y1 = jnp.where(_row_mask(i), y1, 0.0)
    y1_ref[0] = y1[:, :LANES]
    y1_ref[1] = y1[:, LANES:]


def _k3_body(z1_ref, y1_ref, dinv_ref, w2_ref, b2_ref, w3_ref, y2_ref):
    i = pl.program_id(0)
    z = z1_ref[...]
    y1 = y1_ref[...]
    d1 = dinv_ref[...][:, 0:1]
    u = jnp.concatenate([z[0] + y1[0], z[1] + y1[1]], axis=1) * d1
    h2 = jnp.maximum(_dot(u, w2_ref[...]) + b2_ref[...], 0.0)
    t = _dot(h2, w3_ref[...])
    y2 = jnp.where(_row_mask(i), t * d1, 0.0)
    y2_ref[...] = y2


def _k4_body(z2_ref, y2_ref, dinv_ref, b3_ref, out_ref):
    z = z2_ref[...]
    d1 = dinv_ref[...][:, 0:1]
    v = (z[0] + z[1] + y2_ref[...]) * d1 + b3_ref[...]
    m = jnp.max(v, axis=1, keepdims=True)
    lse = jnp.log(jnp.sum(jnp.exp(v - m), axis=1, keepdims=True))
    out_ref[...] = v - m - lse


_GRID = (N_PAD // R_TC,)
_BS_N16 = pl.BlockSpec((R_TC, LANES), lambda i: (i, 0))
_BS_2N16 = pl.BlockSpec((2, R_TC, LANES), lambda i: (0, i, 0))


def _full(shape):
    return pl.BlockSpec(shape, lambda i: tuple(0 for _ in shape))


_deg_kernel = _make_scatter_kernel("deg")
_agg16_kernel = _make_scatter_kernel("agg16")
_agg32_kernel = _make_scatter_kernel("agg32")

_k1 = pl.pallas_call(
    _k1_body, grid=_GRID,
    in_specs=[_BS_2N16, _BS_N16],
    out_specs=[_BS_N16, _BS_N16],
    out_shape=[jax.ShapeDtypeStruct((N_PAD, LANES), jnp.float32)] * 2,
)
_k2 = pl.pallas_call(
    _k2_body, grid=_GRID,
    in_specs=[_BS_2N16, _BS_N16, _BS_N16, _full((LANES, 32)), _full((1, 32))],
    out_specs=_BS_2N16,
    out_shape=jax.ShapeDtypeStruct((2, N_PAD, LANES), jnp.float32),
)
_k3 = pl.pallas_call(
    _k3_body, grid=_GRID,
    in_specs=[_BS_2N16, _BS_2N16, _BS_N16, _full((32, 64)), _full((1, 64)),
              _full((64, LANES))],
    out_specs=_BS_N16,
    out_shape=jax.ShapeDtypeStruct((N_PAD, LANES), jnp.float32),
)
_k4 = pl.pallas_call(
    _k4_body, grid=_GRID,
    in_specs=[_BS_2N16, _BS_N16, _BS_N16, _full((1, LANES))],
    out_specs=_BS_N16,
    out_shape=jax.ShapeDtypeStruct((N_PAD, LANES), jnp.float32),
)


@jax.jit
def kernel(x, edge_index, W1, b1, W2, b2, W3, b3):
    e = edge_index.shape[1]
    src = edge_index[0].astype(jnp.int32)
    dst = edge_index[1].astype(jnp.int32)
    pad = jnp.full((E_ROWS * GR - e,), DUMMY, jnp.int32)
    srcr = jnp.concatenate([src, pad])
    dstr = jnp.concatenate([dst, pad])

    x16 = jnp.pad(x, ((0, N_PAD - N_NODES), (0, LANES - x.shape[1])))
    e0 = jnp.concatenate(
        [jnp.ones((128, 1), jnp.float32),
         jnp.zeros((128, LANES - 1), jnp.float32)], axis=1)
    w1p = jnp.pad(W1, ((0, LANES - W1.shape[0]), (0, 0)))
    w3p = jnp.pad(W3, ((0, 0), (0, LANES - W3.shape[1])))
    b3p = jnp.concatenate([b3, jnp.full((LANES - b3.shape[0],), NEG, jnp.float32)])

    zdeg = _deg_kernel(x16[None], srcr, dstr, e0)
    y0, dinv = _k1(zdeg, x16)
    z0 = _agg16_kernel(y0[None], srcr, dstr, e0)
    y1h = _k2(z0, y0, dinv, w1p, b1.reshape(1, -1))
    z1 = _agg32_kernel(y1h, srcr, dstr, e0)
    y2 = _k3(z1, y1h, dinv, W2, b2.reshape(1, -1), w3p)
    z2 = _agg16_kernel(y2[None], srcr, dstr, e0)
    out = _k4(z2, y2, dinv, b3p.reshape(1, -1))
    return out[:N_NODES, :10]


# speedup vs baseline: 1.8293x; 1.8293x over previous
import functools

import numpy as np

import jax
import jax.numpy as jnp
from jax import lax
from jax.experimental import pallas as pl
from jax.experimental.pallas import tpu as pltpu
from jax.experimental.pallas import tpu_sc as plsc

N_NODES = 100000
N_PAD = 102400
LANES = 16
E_CHUNK = 128
KD = 4
N_GRP = 3125
ZROWS = 1600
NP8 = N_PAD // 8
R_TC = 1600


def _make_scatter_kernel(mode):
    col_split = mode == "agg32"
    n_workers = 16 if col_split else 32
    mesh = plsc.VectorSubcoreMesh(core_axis_name="c", subcore_axis_name="s")

    def body(table, srcr, dstr, e0c, out, src_v, dst_v, dbuf, accum,
             sem_g, sem_s):
        cid = lax.axis_index("c")
        sid = lax.axis_index("s")
        tix = cid if col_split else 0
        for t in range(4):
            pltpu.sync_copy(table.at[tix, pl.ds(N_NODES, ZROWS)],
                            accum.at[pl.ds(sid * 6400 + t * ZROWS, ZROWS)])
        if mode == "deg":
            for j in range(KD):
                pltpu.sync_copy(e0c, dbuf.at[0, j])
        plsc.subcore_barrier()

        wid = sid if col_split else cid * 16 + sid
        g_lo = wid * N_GRP // n_workers
        g_hi = (wid + 1) * N_GRP // n_workers

        def load_idx(b, g):
            base = g * KD
            if mode != "deg":
                pltpu.sync_copy(srcr.at[pl.ds(base, KD)], src_v.at[b])
            pltpu.sync_copy(dstr.at[pl.ds(base, KD)], dst_v.at[b])

        def fire_gathers(b):
            for j in range(KD):
                pltpu.async_copy(table.at[tix].at[src_v.at[b, j]],
                                 dbuf.at[b, j], sem_g)

        def scatter_src(b, j):
            return dbuf.at[0, j] if mode == "deg" else dbuf.at[b, j]

        load_idx(lax.rem(g_lo, 2), g_lo)
        if mode != "deg":
            fire_gathers(lax.rem(g_lo, 2))

        def grp(g, carry):
            b = lax.rem(g, 2)
            nb = 1 - b
            if mode != "deg":
                for j in range(KD):
                    pltpu.make_async_copy(table.at[tix].at[src_v.at[b, j]],
                                          dbuf.at[b, j], sem_g).wait()
            for j in range(KD):
                pltpu.async_copy(scatter_src(b, j),
                                 accum.at[dst_v.at[b, j]], sem_s, add=True)

            @pl.when(g > g_lo)
            def _drain_prev():
                for j in range(KD):
                    pltpu.make_async_copy(scatter_src(nb, j),
                                          accum.at[dst_v.at[nb, j]],
                                          sem_s).wait()

            @pl.when(g + 1 < g_hi)
            def _prefetch():
                load_idx(nb, g + 1)
                if mode != "deg":
                    fire_gathers(nb)

            return carry

        lax.fori_loop(g_lo, g_hi, grp, 0)
        last = lax.rem(g_hi - 1, 2)
        for j in range(KD):
            pltpu.make_async_copy(scatter_src(last, j),
                                  accum.at[dst_v.at[last, j]], sem_s).wait()
        plsc.subcore_barrier()
        for t in range(4):
            sl = pl.ds(sid * 6400 + t * ZROWS, ZROWS)
            pltpu.sync_copy(accum.at[sl], out.at[cid, sl])

    return pl.kernel(
        body,
        out_type=jax.ShapeDtypeStruct((2, N_PAD, LANES), jnp.float32),
        mesh=mesh,
        scratch_types=[
            pltpu.VMEM((2, KD, E_CHUNK), jnp.int32),
            pltpu.VMEM((2, KD, E_CHUNK), jnp.int32),
            pltpu.VMEM((2, KD, E_CHUNK, LANES), jnp.float32),
            pltpu.VMEM_SHARED((N_PAD, LANES), jnp.float32),
            pltpu.SemaphoreType.DMA,
            pltpu.SemaphoreType.DMA,
        ],
        compiler_params=pltpu.CompilerParams(use_tc_tiling_on_sc=False),
    )


_deg_kernel = _make_scatter_kernel("deg")
_agg16_kernel = _make_scatter_kernel("agg16")
_agg32_kernel = _make_scatter_kernel("agg32")


def _np_mask16():
    m = np.zeros((1, 128), np.float32)
    m[0, ::16] = 1.0
    return m


def _np_rep(w):
    m = np.zeros((128, 8 * w), np.float32)
    for g in range(8):
        m[16 * g, w * g:w * (g + 1)] = 1.0
    return m


def _np_mean10():
    m = np.zeros((128, 128), np.float32)
    for g in range(8):
        for i in range(10):
            m[16 * g + i, 16 * g:16 * (g + 1)] = 0.1
    return m


def _np_sum16():
    m = np.zeros((128, 128), np.float32)
    for g in range(8):
        m[16 * g:16 * (g + 1), 16 * g:16 * (g + 1)] = 1.0
    return m


def _np_mask10():
    m = np.zeros((1, 128), np.float32)
    for g in range(8):
        m[0, 16 * g:16 * g + 10] = 1.0
    return m


def _blockdiag(w):
    k, n = w.shape
    out = jnp.zeros((8 * k, 8 * n), w.dtype)
    for g in range(8):
        out = lax.dynamic_update_slice(out, w, (g * k, g * n))
    return out


MASK16 = _np_mask16()
REP16 = _np_rep(16)
MEAN10 = _np_mean10()
SUM16 = _np_sum16()
MASK10 = _np_mask10()


def _dot(a, b):
    return lax.dot_general(a, b, (((1,), (0,)), ((), ())),
                           preferred_element_type=jnp.float32)


def _row_mask(i):
    rid = lax.broadcasted_iota(jnp.int32, (R_TC, 1), 0) + i * R_TC
    return rid < (N_NODES // 8)


def _k1_body(zd_ref, x_ref, m16_ref, rep_ref, y0_ref, dsel_ref):
    zd = zd_ref[...]
    dsel = m16_ref[...] / jnp.sqrt(zd[0] + zd[1] + 1.0)
    y0_ref[...] = x_ref[...] * _dot(dsel, rep_ref[...])
    dsel_ref[...] = dsel


def _k2_body(z0_ref, y0_ref, dsel_ref, rep_ref, w1a_ref, w1b_ref, b1_ref,
             y1_ref):
    i = pl.program_id(0)
    z = z0_ref[...]
    d16 = _dot(dsel_ref[...], rep_ref[...])
    u = (z[0] + z[1] + y0_ref[...]) * d16
    b1 = b1_ref[...]
    mask = _row_mask(i)
    h1a = jnp.maximum(_dot(u, w1a_ref[...]) + b1[:, :128], 0.0)
    h1b = jnp.maximum(_dot(u, w1b_ref[...]) + b1[:, 128:], 0.0)
    y1_ref[0] = jnp.where(mask, h1a * d16, 0.0)
    y1_ref[1] = jnp.where(mask, h1b * d16, 0.0)


def _k3_body(z1_ref, y1_ref, dsel_ref, rep_ref, w2a_ref, w2b_ref, b2_ref,
             w3_ref, y2_ref):
    i = pl.program_id(0)
    z = z1_ref[...]
    y1 = y1_ref[...]
    d16 = _dot(dsel_ref[...], rep_ref[...])
    ua = (z[0] + y1[0]) * d16
    ub = (z[1] + y1[1]) * d16
    h2 = jnp.maximum(_dot(ua, w2a_ref[...]) + _dot(ub, w2b_ref[...])
                     + b2_ref[...], 0.0)
    t = _dot(h2, w3_ref[...])
    y2_ref[...] = jnp.where(_row_mask(i), t * d16, 0.0)


def _k4_body(z2_ref, y2_ref, dsel_ref, rep_ref, mean_ref, sum_ref, m10_ref,
             b3_ref, out_ref):
    z = z2_ref[...]
    d16 = _dot(dsel_ref[...], rep_ref[...])
    v = (z[0] + z[1] + y2_ref[...]) * d16 + b3_ref[...]
    mean = _dot(v, mean_ref[...])
    m10 = m10_ref[...]
    w = (v - mean) * m10 - 100.0 * (1.0 - m10)
    lse = jnp.log(_dot(jnp.exp(w), sum_ref[...]))
    out_ref[...] = v - mean - lse


_GRID = (NP8 // R_TC,)
_BS_P = pl.BlockSpec((R_TC, 128), lambda i: (i, 0))
_BS_2P = pl.BlockSpec((2, R_TC, 128), lambda i: (0, i, 0))


def _full(shape):
    return pl.BlockSpec(shape, lambda i: tuple(0 for _ in shape))


_PF32 = jax.ShapeDtypeStruct((NP8, 128), jnp.float32)

_k1 = pl.pallas_call(
    _k1_body, grid=_GRID,
    in_specs=[_BS_2P, _BS_P, _full((1, 128)), _full((128, 128))],
    out_specs=[_BS_P, _BS_P],
    out_shape=[_PF32, _PF32],
)
_k2 = pl.pallas_call(
    _k2_body, grid=_GRID,
    in_specs=[_BS_2P, _BS_P, _BS_P, _full((128, 128)), _full((128, 128)),
              _full((128, 128)), _full((1, 256))],
    out_specs=_BS_2P,
    out_shape=jax.ShapeDtypeStruct((2, NP8, 128), jnp.float32),
)
_k3 = pl.pallas_call(
    _k3_body, grid=_GRID,
    in_specs=[_BS_2P, _BS_2P, _BS_P, _full((128, 128)), _full((128, 512)),
              _full((128, 512)), _full((1, 512)), _full((512, 128))],
    out_specs=_BS_P,
    out_shape=_PF32,
)
_k4 = pl.pallas_call(
    _k4_body, grid=_GRID,
    in_specs=[_BS_2P, _BS_P, _BS_P, _full((128, 128)), _full((128, 128)),
              _full((128, 128)), _full((1, 128)), _full((1, 128))],
    out_specs=_BS_P,
    out_shape=_PF32,
)


def _packed(a2d):
    return a2d.reshape(2, NP8, 128)


def _tabled(ap):
    return ap.reshape(1, N_PAD, LANES)


@jax.jit
def kernel(x, edge_index, W1, b1, W2, b2, W3, b3):
    src = edge_index[0].astype(jnp.int32).reshape(N_GRP * KD, E_CHUNK)
    dst = edge_index[1].astype(jnp.int32).reshape(N_GRP * KD, E_CHUNK)

    x16 = jnp.pad(x, ((0, N_PAD - N_NODES), (0, LANES - x.shape[1])))
    xp = x16.reshape(NP8, 128)
    e0 = jnp.concatenate(
        [jnp.ones((E_CHUNK, 1), jnp.float32),
         jnp.zeros((E_CHUNK, LANES - 1), jnp.float32)], axis=1)

    w1a = _blockdiag(jnp.pad(W1[:, :16], ((0, 6), (0, 0))))
    w1b = _blockdiag(jnp.pad(W1[:, 16:], ((0, 6), (0, 0))))
    w2a = _blockdiag(W2[:16, :])
    w2b = _blockdiag(W2[16:, :])
    w3blk = _blockdiag(jnp.pad(W3, ((0, 0), (0, 6))))
    b1t = jnp.concatenate([jnp.tile(b1[:16], 8), jnp.tile(b1[16:], 8)])
    b2t = jnp.tile(b2, 8).reshape(1, 512)
    b3t = jnp.tile(jnp.pad(b3, (0, 6)), 8).reshape(1, 128)

    zdeg = _deg_kernel(_tabled(xp), src, dst, e0)
    y0, dsel = _k1(_packed(zdeg), xp, MASK16, REP16)
    z0 = _agg16_kernel(_tabled(y0), src, dst, e0)
    y1h = _k2(_packed(z0), y0, dsel, REP16, w1a, w1b, b1t.reshape(1, 256))
    z1 = _agg32_kernel(y1h.reshape(2, N_PAD, LANES), src, dst, e0)
    y2 = _k3(_packed(z1), y1h, dsel, REP16, w2a, w2b, b2t, w3blk)
    z2 = _agg16_kernel(_tabled(y2), src, dst, e0)
    out = _k4(_packed(z2), y2, dsel, REP16, MEAN10, SUM16, MASK10, b3t)
    return out.reshape(N_PAD, LANES)[:N_NODES, :10]


# speedup vs baseline: 1.8913x; 1.0339x over previous
import functools

import numpy as np

import jax
import jax.numpy as jnp
from jax import lax
from jax.experimental import pallas as pl
from jax.experimental.pallas import tpu as pltpu
from jax.experimental.pallas import tpu_sc as plsc

N_NODES = 100000
N_PAD = 102400
LANES = 16
E_CHUNK = 128
KD = 4
N_GRP = 3125
ZROWS = 1600
NP8 = N_PAD // 8
R_TC = 1600


def _make_scatter_kernel(mode):
    col_split = mode == "agg32"
    n_workers = 16 if col_split else 32
    mesh = plsc.VectorSubcoreMesh(core_axis_name="c", subcore_axis_name="s")

    def body(table, er, e0c, out, src_v, dst_v, dbuf, accum,
             sem_g, sem_s):
        cid = lax.axis_index("c")
        sid = lax.axis_index("s")
        tix = cid if col_split else 0
        for t in range(4):
            pltpu.sync_copy(table.at[tix, pl.ds(N_NODES, ZROWS)],
                            accum.at[pl.ds(sid * 6400 + t * ZROWS, ZROWS)])
        if mode == "deg":
            for j in range(KD):
                pltpu.sync_copy(e0c, dbuf.at[0, j])
        plsc.subcore_barrier()

        wid = sid if col_split else cid * 16 + sid
        g_lo = wid * N_GRP // n_workers
        g_hi = (wid + 1) * N_GRP // n_workers

        def load_idx(b, g):
            base = g * KD
            if mode != "deg":
                pltpu.sync_copy(er.at[0, pl.ds(base, KD)], src_v.at[b])
            pltpu.sync_copy(er.at[1, pl.ds(base, KD)], dst_v.at[b])

        def fire_gathers(b):
            for j in range(KD):
                pltpu.async_copy(table.at[tix].at[src_v.at[b, j]],
                                 dbuf.at[b, j], sem_g)

        def scatter_src(b, j):
            return dbuf.at[0, j] if mode == "deg" else dbuf.at[b, j]

        load_idx(lax.rem(g_lo, 2), g_lo)
        if mode != "deg":
            fire_gathers(lax.rem(g_lo, 2))

        def grp(g, carry):
            b = lax.rem(g, 2)
            nb = 1 - b
            if mode != "deg":
                for j in range(KD):
                    pltpu.make_async_copy(table.at[tix].at[src_v.at[b, j]],
                                          dbuf.at[b, j], sem_g).wait()
            for j in range(KD):
                pltpu.async_copy(scatter_src(b, j),
                                 accum.at[dst_v.at[b, j]], sem_s, add=True)

            @pl.when(g > g_lo)
            def _drain_prev():
                for j in range(KD):
                    pltpu.make_async_copy(scatter_src(nb, j),
                                          accum.at[dst_v.at[nb, j]],
                                          sem_s).wait()

            @pl.when(g + 1 < g_hi)
            def _prefetch():
                load_idx(nb, g + 1)
                if mode != "deg":
                    fire_gathers(nb)

            return carry

        lax.fori_loop(g_lo, g_hi, grp, 0)
        last = lax.rem(g_hi - 1, 2)
        for j in range(KD):
            pltpu.make_async_copy(scatter_src(last, j),
                                  accum.at[dst_v.at[last, j]], sem_s).wait()
        plsc.subcore_barrier()
        for t in range(4):
            sl = pl.ds(sid * 6400 + t * ZROWS, ZROWS)
            pltpu.sync_copy(accum.at[sl], out.at[cid, sl])

    return pl.kernel(
        body,
        out_type=jax.ShapeDtypeStruct((2, N_PAD, LANES), jnp.float32),
        mesh=mesh,
        scratch_types=[
            pltpu.VMEM((2, KD, E_CHUNK), jnp.int32),
            pltpu.VMEM((2, KD, E_CHUNK), jnp.int32),
            pltpu.VMEM((2, KD, E_CHUNK, LANES), jnp.float32),
            pltpu.VMEM_SHARED((N_PAD, LANES), jnp.float32),
            pltpu.SemaphoreType.DMA,
            pltpu.SemaphoreType.DMA,
        ],
        compiler_params=pltpu.CompilerParams(use_tc_tiling_on_sc=False),
    )


_deg_kernel = _make_scatter_kernel("deg")
_agg16_kernel = _make_scatter_kernel("agg16")
_agg32_kernel = _make_scatter_kernel("agg32")


def _np_mask16():
    m = np.zeros((1, 128), np.float32)
    m[0, ::16] = 1.0
    return m


def _np_rep(w):
    m = np.zeros((128, 8 * w), np.float32)
    for g in range(8):
        m[16 * g, w * g:w * (g + 1)] = 1.0
    return m


def _np_mean10():
    m = np.zeros((128, 128), np.float32)
    for g in range(8):
        for i in range(10):
            m[16 * g + i, 16 * g:16 * (g + 1)] = 0.1
    return m


def _np_sum16():
    m = np.zeros((128, 128), np.float32)
    for g in range(8):
        m[16 * g:16 * (g + 1), 16 * g:16 * (g + 1)] = 1.0
    return m


def _np_mask10():
    m = np.zeros((1, 128), np.float32)
    for g in range(8):
        m[0, 16 * g:16 * g + 10] = 1.0
    return m


def _blockdiag(w):
    return jnp.kron(jnp.eye(8, dtype=w.dtype), w)


MASK16 = _np_mask16()
REP16 = _np_rep(16)
MEAN10 = _np_mean10()
SUM16 = _np_sum16()
MASK10 = _np_mask10()


def _dot(a, b):
    return lax.dot_general(a, b, (((1,), (0,)), ((), ())),
                           preferred_element_type=jnp.float32)


def _row_mask(i):
    rid = lax.broadcasted_iota(jnp.int32, (R_TC, 1), 0) + i * R_TC
    return rid < (N_NODES // 8)


def _k1_body(zd_ref, x_ref, m16_ref, rep_ref, y0_ref, dsel_ref):
    zd = zd_ref[...]
    dsel = m16_ref[...] / jnp.sqrt(zd[0] + zd[1] + 1.0)
    y0_ref[...] = x_ref[...] * _dot(dsel, rep_ref[...])
    dsel_ref[...] = dsel


def _k2_body(z0_ref, y0_ref, dsel_ref, rep_ref, w1a_ref, w1b_ref, b1_ref,
             y1_ref):
    i = pl.program_id(0)
    z = z0_ref[...]
    d16 = _dot(dsel_ref[...], rep_ref[...])
    u = (z[0] + z[1] + y0_ref[...]) * d16
    b1 = b1_ref[...]
    mask = _row_mask(i)
    h1a = jnp.maximum(_dot(u, w1a_ref[...]) + b1[:, :128], 0.0)
    h1b = jnp.maximum(_dot(u, w1b_ref[...]) + b1[:, 128:], 0.0)
    y1_ref[0] = jnp.where(mask, h1a * d16, 0.0)
    y1_ref[1] = jnp.where(mask, h1b * d16, 0.0)


def _k3_body(z1_ref, y1_ref, dsel_ref, rep_ref, w2a_ref, w2b_ref, b2_ref,
             w3_ref, y2_ref):
    i = pl.program_id(0)
    z = z1_ref[...]
    y1 = y1_ref[...]
    d16 = _dot(dsel_ref[...], rep_ref[...])
    ua = (z[0] + y1[0]) * d16
    ub = (z[1] + y1[1]) * d16
    h2 = jnp.maximum(_dot(ua, w2a_ref[...]) + _dot(ub, w2b_ref[...])
                     + b2_ref[...], 0.0)
    t = _dot(h2, w3_ref[...])
    y2_ref[...] = jnp.where(_row_mask(i), t * d16, 0.0)


def _k4_body(z2_ref, y2_ref, dsel_ref, rep_ref, mean_ref, sum_ref, m10_ref,
             b3_ref, out_ref):
    z = z2_ref[...]
    d16 = _dot(dsel_ref[...], rep_ref[...])
    v = (z[0] + z[1] + y2_ref[...]) * d16 + b3_ref[...]
    mean = _dot(v, mean_ref[...])
    m10 = m10_ref[...]
    w = (v - mean) * m10 - 100.0 * (1.0 - m10)
    lse = jnp.log(_dot(jnp.exp(w), sum_ref[...]))
    out_ref[...] = v - mean - lse


_GRID = (NP8 // R_TC,)
_BS_P = pl.BlockSpec((R_TC, 128), lambda i: (i, 0))
_BS_2P = pl.BlockSpec((2, R_TC, 128), lambda i: (0, i, 0))


def _full(shape):
    return pl.BlockSpec(shape, lambda i: tuple(0 for _ in shape))


_PF32 = jax.ShapeDtypeStruct((NP8, 128), jnp.float32)

_k1 = pl.pallas_call(
    _k1_body, grid=_GRID,
    in_specs=[_BS_2P, _BS_P, _full((1, 128)), _full((128, 128))],
    out_specs=[_BS_P, _BS_P],
    out_shape=[_PF32, _PF32],
)
_k2 = pl.pallas_call(
    _k2_body, grid=_GRID,
    in_specs=[_BS_2P, _BS_P, _BS_P, _full((128, 128)), _full((128, 128)),
              _full((128, 128)), _full((1, 256))],
    out_specs=_BS_2P,
    out_shape=jax.ShapeDtypeStruct((2, NP8, 128), jnp.float32),
)
_k3 = pl.pallas_call(
    _k3_body, grid=_GRID,
    in_specs=[_BS_2P, _BS_2P, _BS_P, _full((128, 128)), _full((128, 512)),
              _full((128, 512)), _full((1, 512)), _full((512, 128))],
    out_specs=_BS_P,
    out_shape=_PF32,
)
_k4 = pl.pallas_call(
    _k4_body, grid=_GRID,
    in_specs=[_BS_2P, _BS_P, _BS_P, _full((128, 128)), _full((128, 128)),
              _full((128, 128)), _full((1, 128)), _full((1, 128))],
    out_specs=_BS_P,
    out_shape=_PF32,
)


def _packed(a2d):
    return a2d.reshape(2, NP8, 128)


def _tabled(ap):
    return ap.reshape(1, N_PAD, LANES)


@jax.jit
def kernel(x, edge_index, W1, b1, W2, b2, W3, b3):
    er = edge_index.astype(jnp.int32).reshape(2, N_GRP * KD, E_CHUNK)

    x16 = jnp.pad(x, ((0, N_PAD - N_NODES), (0, LANES - x.shape[1])))
    xp = x16.reshape(NP8, 128)
    e0 = jnp.concatenate(
        [jnp.ones((E_CHUNK, 1), jnp.float32),
         jnp.zeros((E_CHUNK, LANES - 1), jnp.float32)], axis=1)

    w1a = _blockdiag(jnp.pad(W1[:, :16], ((0, 6), (0, 0))))
    w1b = _blockdiag(jnp.pad(W1[:, 16:], ((0, 6), (0, 0))))
    w2a = _blockdiag(W2[:16, :])
    w2b = _blockdiag(W2[16:, :])
    w3blk = _blockdiag(jnp.pad(W3, ((0, 0), (0, 6))))
    b1t = jnp.concatenate([jnp.tile(b1[:16], 8), jnp.tile(b1[16:], 8)])
    b2t = jnp.tile(b2, 8).reshape(1, 512)
    b3t = jnp.tile(jnp.pad(b3, (0, 6)), 8).reshape(1, 128)

    zdeg = _deg_kernel(_tabled(xp), er, e0)
    y0, dsel = _k1(_packed(zdeg), xp, MASK16, REP16)
    z0 = _agg16_kernel(_tabled(y0), er, e0)
    y1h = _k2(_packed(z0), y0, dsel, REP16, w1a, w1b, b1t.reshape(1, 256))
    z1 = _agg32_kernel(y1h.reshape(2, N_PAD, LANES), er, e0)
    y2 = _k3(_packed(z1), y1h, dsel, REP16, w2a, w2b, b2t, w3blk)
    z2 = _agg16_kernel(_tabled(y2), er, e0)
    out = _k4(_packed(z2), y2, dsel, REP16, MEAN10, SUM16, MASK10, b3t)
    return out.reshape(N_PAD, LANES)[:N_NODES, :10]


# speedup vs baseline: 1.8964x; 1.0027x over previous
import functools

import numpy as np

import jax
import jax.numpy as jnp
from jax import lax
from jax.experimental import pallas as pl
from jax.experimental.pallas import tpu as pltpu
from jax.experimental.pallas import tpu_sc as plsc

N_NODES = 100000
N_PAD = 102400
LANES = 16
E_CHUNK = 128
KD = 4
N_GRP = 3125
ZROWS = 1600
NP8 = N_PAD // 8
R_TC = 1600


def _make_scatter_kernel(mode):
    col_split = mode == "agg32"
    n_workers = 16 if col_split else 32
    mesh = plsc.VectorSubcoreMesh(core_axis_name="c", subcore_axis_name="s")

    def body(table, er, e0c, zz, out, src_v, dst_v, dbuf, accum,
             sem_g, sem_s):
        cid = lax.axis_index("c")
        sid = lax.axis_index("s")
        tix = cid if col_split else 0
        for t in range(4):
            pltpu.sync_copy(zz,
                            accum.at[pl.ds(sid * 6400 + t * ZROWS, ZROWS)])
        if mode == "deg":
            for j in range(KD):
                pltpu.sync_copy(e0c, dbuf.at[0, j])
        plsc.subcore_barrier()

        wid = sid if col_split else cid * 16 + sid
        g_lo = wid * N_GRP // n_workers
        g_hi = (wid + 1) * N_GRP // n_workers

        def load_idx(b, g):
            base = g * KD
            if mode != "deg":
                pltpu.sync_copy(er.at[0, pl.ds(base, KD)], src_v.at[b])
            pltpu.sync_copy(er.at[1, pl.ds(base, KD)], dst_v.at[b])

        def fire_gathers(b):
            for j in range(KD):
                pltpu.async_copy(table.at[tix].at[src_v.at[b, j]],
                                 dbuf.at[b, j], sem_g)

        def scatter_src(b, j):
            return dbuf.at[0, j] if mode == "deg" else dbuf.at[b, j]

        load_idx(lax.rem(g_lo, 2), g_lo)
        if mode != "deg":
            fire_gathers(lax.rem(g_lo, 2))

        def grp(g, carry):
            b = lax.rem(g, 2)
            nb = 1 - b
            if mode != "deg":
                for j in range(KD):
                    pltpu.make_async_copy(table.at[tix].at[src_v.at[b, j]],
                                          dbuf.at[b, j], sem_g).wait()
            for j in range(KD):
                pltpu.async_copy(scatter_src(b, j),
                                 accum.at[dst_v.at[b, j]], sem_s, add=True)

            @pl.when(g > g_lo)
            def _drain_prev():
                for j in range(KD):
                    pltpu.make_async_copy(scatter_src(nb, j),
                                          accum.at[dst_v.at[nb, j]],
                                          sem_s).wait()

            @pl.when(g + 1 < g_hi)
            def _prefetch():
                load_idx(nb, g + 1)
                if mode != "deg":
                    fire_gathers(nb)

            return carry

        lax.fori_loop(g_lo, g_hi, grp, 0)
        last = lax.rem(g_hi - 1, 2)
        for j in range(KD):
            pltpu.make_async_copy(scatter_src(last, j),
                                  accum.at[dst_v.at[last, j]], sem_s).wait()
        plsc.subcore_barrier()
        for t in range(4):
            sl = pl.ds(sid * 6400 + t * ZROWS, ZROWS)
            pltpu.sync_copy(accum.at[sl], out.at[cid, sl])

    return pl.kernel(
        body,
        out_type=jax.ShapeDtypeStruct((2, N_PAD, LANES), jnp.float32),
        mesh=mesh,
        scratch_types=[
            pltpu.VMEM((2, KD, E_CHUNK), jnp.int32),
            pltpu.VMEM((2, KD, E_CHUNK), jnp.int32),
            pltpu.VMEM((2, KD, E_CHUNK, LANES), jnp.float32),
            pltpu.VMEM_SHARED((N_PAD, LANES), jnp.float32),
            pltpu.SemaphoreType.DMA,
            pltpu.SemaphoreType.DMA,
        ],
        compiler_params=pltpu.CompilerParams(use_tc_tiling_on_sc=False),
    )


_deg_kernel = _make_scatter_kernel("deg")
_agg16_kernel = _make_scatter_kernel("agg16")
_agg32_kernel = _make_scatter_kernel("agg32")


def _np_mask16():
    m = np.zeros((1, 128), np.float32)
    m[0, ::16] = 1.0
    return m


def _np_rep(w):
    m = np.zeros((128, 8 * w), np.float32)
    for g in range(8):
        m[16 * g, w * g:w * (g + 1)] = 1.0
    return m


def _np_mean10():
    m = np.zeros((128, 128), np.float32)
    for g in range(8):
        for i in range(10):
            m[16 * g + i, 16 * g:16 * (g + 1)] = 0.1
    return m


def _np_sum16():
    m = np.zeros((128, 128), np.float32)
    for g in range(8):
        m[16 * g:16 * (g + 1), 16 * g:16 * (g + 1)] = 1.0
    return m


def _np_mask10():
    m = np.zeros((1, 128), np.float32)
    for g in range(8):
        m[0, 16 * g:16 * g + 10] = 1.0
    return m


def _blockdiag(w):
    return jnp.kron(jnp.eye(8, dtype=w.dtype), w)


MASK16 = _np_mask16()
REP16 = _np_rep(16)
MEAN10 = _np_mean10()
SUM16 = _np_sum16()
MASK10 = _np_mask10()


def _dot(a, b):
    return lax.dot_general(a, b, (((1,), (0,)), ((), ())),
                           preferred_element_type=jnp.float32)


def _row_mask(i):
    rid = lax.broadcasted_iota(jnp.int32, (R_TC, 1), 0) + i * R_TC
    return rid < (N_NODES // 8)


def _k1_body(zd_ref, x_ref, m16_ref, rep_ref, y0_ref, dsel_ref):
    zd = zd_ref[...]
    dsel = m16_ref[...] / jnp.sqrt(zd[0] + zd[1] + 1.0)
    y0_ref[...] = x_ref[...] * _dot(dsel, rep_ref[...])
    dsel_ref[...] = dsel


def _k2_body(z0_ref, y0_ref, dsel_ref, rep_ref, w1a_ref, w1b_ref, b1_ref,
             y1_ref):
    i = pl.program_id(0)
    z = z0_ref[...]
    d16 = _dot(dsel_ref[...], rep_ref[...])
    u = (z[0] + z[1] + y0_ref[...]) * d16
    b1 = b1_ref[...]
    mask = _row_mask(i)
    h1a = jnp.maximum(_dot(u, w1a_ref[...]) + b1[:, :128], 0.0)
    h1b = jnp.maximum(_dot(u, w1b_ref[...]) + b1[:, 128:], 0.0)
    y1_ref[0] = jnp.where(mask, h1a * d16, 0.0)
    y1_ref[1] = jnp.where(mask, h1b * d16, 0.0)


def _k3_body(z1_ref, y1_ref, dsel_ref, rep_ref, w2a_ref, w2b_ref, b2_ref,
             w3_ref, y2_ref):
    i = pl.program_id(0)
    z = z1_ref[...]
    y1 = y1_ref[...]
    d16 = _dot(dsel_ref[...], rep_ref[...])
    ua = (z[0] + y1[0]) * d16
    ub = (z[1] + y1[1]) * d16
    h2 = jnp.maximum(_dot(ua, w2a_ref[...]) + _dot(ub, w2b_ref[...])
                     + b2_ref[...], 0.0)
    t = _dot(h2, w3_ref[...])
    y2_ref[...] = jnp.where(_row_mask(i), t * d16, 0.0)


def _k4_body(z2_ref, y2_ref, dsel_ref, rep_ref, mean_ref, sum_ref, m10_ref,
             b3_ref, out_ref):
    z = z2_ref[...]
    d16 = _dot(dsel_ref[...], rep_ref[...])
    v = (z[0] + z[1] + y2_ref[...]) * d16 + b3_ref[...]
    mean = _dot(v, mean_ref[...])
    m10 = m10_ref[...]
    w = (v - mean) * m10 - 100.0 * (1.0 - m10)
    lse = jnp.log(_dot(jnp.exp(w), sum_ref[...]))
    out_ref[...] = v - mean - lse


_GRID = (NP8 // R_TC,)
_BS_P = pl.BlockSpec((R_TC, 128), lambda i: (i, 0))
_BS_2P = pl.BlockSpec((2, R_TC, 128), lambda i: (0, i, 0))


def _full(shape):
    return pl.BlockSpec(shape, lambda i: tuple(0 for _ in shape))


_PF32 = jax.ShapeDtypeStruct((NP8, 128), jnp.float32)

_k1 = pl.pallas_call(
    _k1_body, grid=_GRID,
    in_specs=[_BS_2P, _BS_P, _full((1, 128)), _full((128, 128))],
    out_specs=[_BS_P, _BS_P],
    out_shape=[_PF32, _PF32],
)
_k2 = pl.pallas_call(
    _k2_body, grid=_GRID,
    in_specs=[_BS_2P, _BS_P, _BS_P, _full((128, 128)), _full((128, 128)),
              _full((128, 128)), _full((1, 256))],
    out_specs=_BS_2P,
    out_shape=jax.ShapeDtypeStruct((2, NP8, 128), jnp.float32),
)
_k3 = pl.pallas_call(
    _k3_body, grid=_GRID,
    in_specs=[_BS_2P, _BS_2P, _BS_P, _full((128, 128)), _full((128, 512)),
              _full((128, 512)), _full((1, 512)), _full((512, 128))],
    out_specs=_BS_P,
    out_shape=_PF32,
)
_k4 = pl.pallas_call(
    _k4_body, grid=_GRID,
    in_specs=[_BS_2P, _BS_P, _BS_P, _full((128, 128)), _full((128, 128)),
              _full((128, 128)), _full((1, 128)), _full((1, 128))],
    out_specs=_BS_P,
    out_shape=_PF32,
)


def _packed(a2d):
    return a2d.reshape(2, NP8, 128)


def _tabled(ap):
    return ap.reshape(1, N_PAD, LANES)


@jax.jit
def kernel(x, edge_index, W1, b1, W2, b2, W3, b3):
    er = edge_index.astype(jnp.int32).reshape(2, N_GRP * KD, E_CHUNK)

    xp = jnp.pad(
        jnp.pad(x.reshape(N_NODES // 8, 8, 10),
                ((0, 0), (0, 0), (0, 6))).reshape(N_NODES // 8, 128),
        ((0, NP8 - N_NODES // 8), (0, 0)))
    zz = jnp.zeros((ZROWS, LANES), jnp.float32)
    e0 = jnp.concatenate(
        [jnp.ones((E_CHUNK, 1), jnp.float32),
         jnp.zeros((E_CHUNK, LANES - 1), jnp.float32)], axis=1)

    w1a = _blockdiag(jnp.pad(W1[:, :16], ((0, 6), (0, 0))))
    w1b = _blockdiag(jnp.pad(W1[:, 16:], ((0, 6), (0, 0))))
    w2a = _blockdiag(W2[:16, :])
    w2b = _blockdiag(W2[16:, :])
    w3blk = _blockdiag(jnp.pad(W3, ((0, 0), (0, 6))))
    b1t = jnp.concatenate([jnp.tile(b1[:16], 8), jnp.tile(b1[16:], 8)])
    b2t = jnp.tile(b2, 8).reshape(1, 512)
    b3t = jnp.tile(jnp.pad(b3, (0, 6)), 8).reshape(1, 128)

    zdeg = _deg_kernel(e0.reshape(1, E_CHUNK, LANES), er, e0, zz)
    y0, dsel = _k1(_packed(zdeg), xp, MASK16, REP16)
    z0 = _agg16_kernel(_tabled(y0), er, e0, zz)
    y1h = _k2(_packed(z0), y0, dsel, REP16, w1a, w1b, b1t.reshape(1, 256))
    z1 = _agg32_kernel(y1h.reshape(2, N_PAD, LANES), er, e0, zz)
    y2 = _k3(_packed(z1), y1h, dsel, REP16, w2a, w2b, b2t, w3blk)
    z2 = _agg16_kernel(_tabled(y2), er, e0, zz)
    out = _k4(_packed(z2), y2, dsel, REP16, MEAN10, SUM16, MASK10, b3t)
    return out.reshape(N_PAD, LANES)[:N_NODES, :10]
